# baseline (speedup 1.0000x reference)
import functools

import jax
import jax.numpy as jnp
from jax import lax
from jax.experimental import pallas as pl
from jax.experimental.pallas import tpu as pltpu
from jax.experimental.pallas import tpu_sc as plsc

NC = 2
NS = 16
NW = NC * NS
CH = 128
K = 4


@functools.partial(jax.jit, static_argnames=())
def _embed_lookup(table, idx3):
    NWv, NCH, CHv = idx3.shape
    V, D = table.shape
    mesh = plsc.VectorSubcoreMesh(
        core_axis_name="c", subcore_axis_name="s", num_cores=NC, num_subcores=NS
    )

    @functools.partial(
        pl.kernel,
        out_type=jax.ShapeDtypeStruct((NWv, NCH, CHv, D), jnp.float32),
        mesh=mesh,
        scratch_types=[
            pltpu.VMEM((NCH, CHv), jnp.int32),
            pltpu.VMEM((K, CHv, D), jnp.float32),
            pltpu.SemaphoreType.DMA,
        ],
        compiler_params=pltpu.CompilerParams(use_tc_tiling_on_sc=False),
    )
    def body(table_hbm, idx_hbm, out_hbm, idx_v, rows_v, sem):
        wid = lax.axis_index("s") * NC + lax.axis_index("c")
        pltpu.sync_copy(idx_hbm.at[wid], idx_v)

        def step_fn(step, carry):
            cps = [
                pltpu.async_copy(
                    table_hbm.at[idx_v.at[step * K + j]], rows_v.at[j], sem
                )
                for j in range(K)
            ]
            for cp in cps:
                cp.wait()
            pltpu.sync_copy(rows_v, out_hbm.at[wid, pl.ds(step * K, K)])
            return carry

        lax.fori_loop(0, NCH // K, step_fn, 0)

    return body(table, idx3)


def kernel(input, table):
    B, S = input.shape
    V, D = table.shape
    n = B * S
    per_w = n // NW
    nch = per_w // CH
    idx3 = input.reshape(NW, nch, CH).astype(jnp.int32)
    out = _embed_lookup(table, idx3)
    return out.reshape(B, S, D)


# speedup vs baseline: 1.0232x; 1.0232x over previous
import functools

import jax
import jax.numpy as jnp
from jax import lax
from jax.experimental import pallas as pl
from jax.experimental.pallas import tpu as pltpu
from jax.experimental.pallas import tpu_sc as plsc

NC = 2
NS = 16
NW = NC * NS
CH = 128
K = 4


@jax.jit
def _embed_lookup(table, idx3):
    NWv, NCH, CHv = idx3.shape
    V, D = table.shape
    nstep = NCH // K
    pairs = nstep // 2
    mesh = plsc.VectorSubcoreMesh(
        core_axis_name="c", subcore_axis_name="s", num_cores=NC, num_subcores=NS
    )

    @functools.partial(
        pl.kernel,
        out_type=jax.ShapeDtypeStruct((NWv, NCH, CHv, D), jnp.float32),
        mesh=mesh,
        scratch_types=[
            pltpu.VMEM((NCH, CHv), jnp.int32),
            pltpu.VMEM((2, K, CHv, D), jnp.float32),
            pltpu.SemaphoreType.DMA,
            pltpu.SemaphoreType.DMA,
        ],
        compiler_params=pltpu.CompilerParams(use_tc_tiling_on_sc=False),
    )
    def body(table_hbm, idx_hbm, out_hbm, idx_v, rows_v, gsem, osem):
        wid = lax.axis_index("s") * NC + lax.axis_index("c")
        pltpu.sync_copy(idx_hbm.at[wid], idx_v)

        def fire(b, step):
            for j in range(K):
                pltpu.async_copy(
                    table_hbm.at[idx_v.at[step * K + j]], rows_v.at[b, j], gsem
                )

        def wait_gathers(b, step):
            for j in range(K):
                pltpu.make_async_copy(
                    table_hbm.at[idx_v.at[step * K + j]], rows_v.at[b, j], gsem
                ).wait()

        def start_out(b, step):
            pltpu.async_copy(
                rows_v.at[b], out_hbm.at[wid, pl.ds(step * K, K)], osem
            )

        def wait_out(b, step):
            pltpu.make_async_copy(
                rows_v.at[b], out_hbm.at[wid, pl.ds(step * K, K)], osem
            ).wait()

        fire(0, 0)

        def macro(m, carry):
            s0 = 2 * m
            s1 = s0 + 1
            wait_gathers(0, s0)
            start_out(0, s0)

            @pl.when(m >= 1)
            def _():
                wait_out(1, s0 - 1)

            fire(1, s1)
            wait_gathers(1, s1)
            start_out(1, s1)

            @pl.when(m + 1 < pairs)
            def _():
                wait_out(0, s1 - 1)
                fire(0, s1 + 1)

            return carry

        lax.fori_loop(0, pairs, macro, 0)
        wait_out(0, nstep - 2)
        wait_out(1, nstep - 1)

    return body(table, idx3)


def kernel(input, table):
    B, S = input.shape
    V, D = table.shape
    n = B * S
    per_w = n // NW
    nch = per_w // CH
    idx3 = input.reshape(NW, nch, CH).astype(jnp.int32)
    out = _embed_lookup(table, idx3)
    return out.reshape(B, S, D)


# speedup vs baseline: 1.0271x; 1.0038x over previous
import functools

import jax
import jax.numpy as jnp
from jax import lax
from jax.experimental import pallas as pl
from jax.experimental.pallas import tpu as pltpu
from jax.experimental.pallas import tpu_sc as plsc

NC = 2
NS = 16
NW = NC * NS
CH = 128
K = 4
NB = 3


@jax.jit
def _embed_lookup(table, idx3):
    NWv, NCH, CHv = idx3.shape
    V, D = table.shape
    nstep = NCH // K
    mesh = plsc.VectorSubcoreMesh(
        core_axis_name="c", subcore_axis_name="s", num_cores=NC, num_subcores=NS
    )

    @functools.partial(
        pl.kernel,
        out_type=jax.ShapeDtypeStruct((NWv, NCH, CHv, D), jnp.float32),
        mesh=mesh,
        scratch_types=[
            pltpu.VMEM((NCH, CHv), jnp.int32),
            pltpu.VMEM((NB, K, CHv, D), jnp.float32),
            pltpu.SemaphoreType.DMA,
            pltpu.SemaphoreType.DMA,
        ],
        compiler_params=pltpu.CompilerParams(use_tc_tiling_on_sc=False),
    )
    def body(table_hbm, idx_hbm, out_hbm, idx_v, rows_v, gsem, osem):
        wid = lax.axis_index("s") * NC + lax.axis_index("c")
        pltpu.sync_copy(idx_hbm.at[wid], idx_v)

        def fire(b, step):
            for j in range(K):
                pltpu.async_copy(
                    table_hbm.at[idx_v.at[step * K + j]], rows_v.at[b, j], gsem
                )

        def wait_gathers(b, step):
            for j in range(K):
                pltpu.make_async_copy(
                    table_hbm.at[idx_v.at[step * K + j]], rows_v.at[b, j], gsem
                ).wait()

        def start_out(b, step):
            pltpu.async_copy(
                rows_v.at[b], out_hbm.at[wid, pl.ds(step * K, K)], osem
            )

        def wait_out(b, step):
            pltpu.make_async_copy(
                rows_v.at[b], out_hbm.at[wid, pl.ds(step * K, K)], osem
            ).wait()

        for s in range(NB - 1):
            fire(s, s)

        def step_fn(s, carry):
            b = lax.rem(s, NB)
            wait_gathers(b, s)
            start_out(b, s)
            s2 = s + (NB - 1)
            b2 = lax.rem(s2, NB)

            @pl.when(s2 < nstep)
            def _():
                @pl.when(s >= 1)
                def _():
                    wait_out(b2, s - 1)

                fire(b2, s2)

            return carry

        lax.fori_loop(0, nstep, step_fn, 0)
        for s in range(nstep - NB, nstep):
            wait_out(s % NB, s)

    return body(table, idx3)


def kernel(input, table):
    B, S = input.shape
    V, D = table.shape
    n = B * S
    per_w = n // NW
    nch = per_w // CH
    idx3 = input.reshape(NW, nch, CH).astype(jnp.int32)
    out = _embed_lookup(table, idx3)
    return out.reshape(B, S, D)


# speedup vs baseline: 1.0275x; 1.0003x over previous
import functools

import jax
import jax.numpy as jnp
from jax import lax
from jax.experimental import pallas as pl
from jax.experimental.pallas import tpu as pltpu
from jax.experimental.pallas import tpu_sc as plsc

NC = 2
NS = 16
NW = NC * NS
G = 2
NB = 3


@jax.jit
def _embed_lookup(table, idx):
    R, S = idx.shape
    V, D = table.shape
    RW = R // NW
    nstep = RW // G
    chunks = [(0, 128), (128, S - 128)] if S > 128 else [(0, S)]
    mesh = plsc.VectorSubcoreMesh(
        core_axis_name="c", subcore_axis_name="s", num_cores=NC, num_subcores=NS
    )

    @functools.partial(
        pl.kernel,
        out_type=jax.ShapeDtypeStruct((R, S, D), jnp.float32),
        mesh=mesh,
        scratch_types=[
            pltpu.VMEM((RW, S), jnp.int32),
            pltpu.VMEM((NB, G, S, D), jnp.float32),
            pltpu.SemaphoreType.DMA,
            pltpu.SemaphoreType.DMA,
        ],
        compiler_params=pltpu.CompilerParams(use_tc_tiling_on_sc=False),
    )
    def body(table_hbm, idx_hbm, out_hbm, idx_v, rows_v, gsem, osem):
        wid = lax.axis_index("s") * NC + lax.axis_index("c")
        row0 = wid * RW
        pltpu.sync_copy(idx_hbm.at[pl.ds(row0, RW)], idx_v)

        def gather_descs(b, step, make):
            out = []
            for g in range(G):
                r = step * G + g
                for (o, w) in chunks:
                    out.append(make(
                        table_hbm.at[idx_v.at[r, pl.ds(o, w)]],
                        rows_v.at[b, g, pl.ds(o, w)],
                        gsem,
                    ))
            return out

        def fire(b, step):
            gather_descs(b, step, pltpu.async_copy)

        def wait_gathers(b, step):
            for d in gather_descs(b, step, pltpu.make_async_copy):
                d.wait()

        def start_out(b, step):
            pltpu.async_copy(
                rows_v.at[b], out_hbm.at[pl.ds(row0 + step * G, G)], osem
            )

        def wait_out(b, step):
            pltpu.make_async_copy(
                rows_v.at[b], out_hbm.at[pl.ds(row0 + step * G, G)], osem
            ).wait()

        for s in range(NB - 1):
            fire(s, s)

        def step_fn(s, carry):
            b = lax.rem(s, NB)
            wait_gathers(b, s)
            start_out(b, s)
            s2 = s + (NB - 1)
            b2 = lax.rem(s2, NB)

            @pl.when(s2 < nstep)
            def _():
                @pl.when(s >= 1)
                def _():
                    wait_out(b2, s - 1)

                fire(b2, s2)

            return carry

        lax.fori_loop(0, nstep, step_fn, 0)
        for s in range(nstep - NB, nstep):
            wait_out(s % NB, s)

    return body(table, idx)


def kernel(input, table):
    return _embed_lookup(table, input.astype(jnp.int32))


# speedup vs baseline: 1.2510x; 1.2175x over previous
import functools

import jax
import jax.numpy as jnp
from jax import lax
from jax.experimental import pallas as pl
from jax.experimental.pallas import tpu as pltpu
from jax.experimental.pallas import tpu_sc as plsc

NC = 2
NS = 16
NW = NC * NS
NB = 3
DP = 128


@jax.jit
def _embed_lookup(table, idx):
    R, S = idx.shape
    V, _ = table.shape
    RW = R // NW
    nstep = RW
    chunks = [(0, 128), (128, S - 128)] if S > 128 else [(0, S)]
    mesh = plsc.VectorSubcoreMesh(
        core_axis_name="c", subcore_axis_name="s", num_cores=NC, num_subcores=NS
    )

    @functools.partial(
        pl.kernel,
        out_type=jax.ShapeDtypeStruct((R, S, DP), jnp.float32),
        mesh=mesh,
        scratch_types=[
            pltpu.VMEM((RW, S), jnp.int32),
            pltpu.VMEM((NB, S, DP), jnp.float32),
            pltpu.SemaphoreType.DMA,
            pltpu.SemaphoreType.DMA,
        ],
        compiler_params=pltpu.CompilerParams(use_tc_tiling_on_sc=False),
    )
    def body(table_hbm, idx_hbm, out_hbm, idx_v, rows_v, gsem, osem):
        wid = lax.axis_index("s") * NC + lax.axis_index("c")
        row0 = wid * RW
        pltpu.sync_copy(idx_hbm.at[pl.ds(row0, RW)], idx_v)

        def gather_descs(b, r, make):
            return [
                make(
                    table_hbm.at[idx_v.at[r, pl.ds(o, w)]],
                    rows_v.at[b, pl.ds(o, w)],
                    gsem,
                )
                for (o, w) in chunks
            ]

        def fire(b, r):
            gather_descs(b, r, pltpu.async_copy)

        def wait_gathers(b, r):
            for d in gather_descs(b, r, pltpu.make_async_copy):
                d.wait()

        def start_out(b, r):
            pltpu.async_copy(rows_v.at[b], out_hbm.at[row0 + r], osem)

        def wait_out(b, r):
            pltpu.make_async_copy(rows_v.at[b], out_hbm.at[row0 + r], osem).wait()

        for s in range(NB - 1):
            fire(s, s)

        def step_fn(s, carry):
            b = lax.rem(s, NB)
            wait_gathers(b, s)
            start_out(b, s)
            s2 = s + (NB - 1)
            b2 = lax.rem(s2, NB)

            @pl.when(s2 < nstep)
            def _():
                @pl.when(s >= 1)
                def _():
                    wait_out(b2, s - 1)

                fire(b2, s2)

            return carry

        lax.fori_loop(0, nstep, step_fn, 0)
        for s in range(nstep - NB, nstep):
            wait_out(s % NB, s)

    return body(table, idx)


def kernel(input, table):
    table128 = jnp.pad(table, ((0, 0), (0, DP - table.shape[1])))
    out128 = _embed_lookup(table128, input.astype(jnp.int32))
    return out128[:, :, : table.shape[1]]


# speedup vs baseline: 1.3431x; 1.0737x over previous
import functools

import jax
import jax.numpy as jnp
from jax import lax
from jax.experimental import pallas as pl
from jax.experimental.pallas import tpu as pltpu
from jax.experimental.pallas import tpu_sc as plsc

NC = 2
NS = 16
NW = NC * NS
NB = 3
DP = 128


@jax.jit
def _embed_lookup(table, idx):
    R, S = idx.shape
    V, _ = table.shape
    RW = R // NW
    nstep = RW
    chunks = [(0, 128), (128, S - 128)] if S > 128 else [(0, S)]
    mesh = plsc.VectorSubcoreMesh(
        core_axis_name="c", subcore_axis_name="s", num_cores=NC, num_subcores=NS
    )

    @functools.partial(
        pl.kernel,
        out_type=jax.ShapeDtypeStruct((R, S, DP), jnp.float32),
        mesh=mesh,
        scratch_types=[
            pltpu.VMEM((RW, S), jnp.int32),
            pltpu.VMEM((NB, S, DP), jnp.float32),
            pltpu.SemaphoreType.DMA,
            pltpu.SemaphoreType.DMA,
        ],
        compiler_params=pltpu.CompilerParams(use_tc_tiling_on_sc=False),
    )
    def body(table_hbm, idx_hbm, out_hbm, idx_v, rows_v, gsem, osem):
        wid = lax.axis_index("s") * NC + lax.axis_index("c")
        row0 = wid * RW
        pltpu.sync_copy(idx_hbm.at[pl.ds(row0, RW)], idx_v)

        def gather_descs(b, r, make):
            return [
                make(
                    table_hbm.at[idx_v.at[r, pl.ds(o, w)]],
                    rows_v.at[b, pl.ds(o, w)],
                    gsem,
                )
                for (o, w) in chunks
            ]

        def fire(b, r):
            gather_descs(b, r, pltpu.async_copy)

        def wait_gathers(b, r):
            for d in gather_descs(b, r, pltpu.make_async_copy):
                d.wait()

        def start_out(b, r):
            pltpu.async_copy(
                rows_v.at[b, :, pl.ds(0, 64)],
                out_hbm.at[row0 + r, :, pl.ds(0, 64)],
                osem,
            )

        def wait_out(b, r):
            pltpu.make_async_copy(
                rows_v.at[b, :, pl.ds(0, 64)],
                out_hbm.at[row0 + r, :, pl.ds(0, 64)],
                osem,
            ).wait()

        for s in range(NB - 1):
            fire(s, s)

        def step_fn(s, carry):
            b = lax.rem(s, NB)
            wait_gathers(b, s)
            start_out(b, s)
            s2 = s + (NB - 1)
            b2 = lax.rem(s2, NB)

            @pl.when(s2 < nstep)
            def _():
                @pl.when(s >= 1)
                def _():
                    wait_out(b2, s - 1)

                fire(b2, s2)

            return carry

        lax.fori_loop(0, nstep, step_fn, 0)
        for s in range(nstep - NB, nstep):
            wait_out(s % NB, s)

    return body(table, idx)


def kernel(input, table):
    table128 = jnp.pad(table, ((0, 0), (0, DP - table.shape[1])))
    out128 = _embed_lookup(table128, input.astype(jnp.int32))
    return out128[:, :, : table.shape[1]]
